# Initial kernel scaffold; baseline (speedup 1.0000x reference)
import jax
import jax.numpy as jnp
from jax.experimental import pallas as pl


def kernel(x, table):
    raise NotImplementedError("write your pallas kernel here")


# speedup vs baseline: 2.8328x; 2.8328x over previous
import functools

import jax
import jax.numpy as jnp
import numpy as np
from jax import lax
from jax.experimental import pallas as pl
from jax.experimental.pallas import tpu as pltpu
from jax.experimental.pallas import tpu_sc as plsc


def _positional_encoding(length: int, depth: int) -> np.ndarray:
    half = depth // 2
    positions = np.arange(length)[:, np.newaxis]
    depths = np.arange(half)[np.newaxis, :] / half
    angle_rates = 1.0 / (10000.0 ** depths)
    angle_rads = positions * angle_rates
    return np.concatenate(
        [np.sin(angle_rads), np.cos(angle_rads)], axis=-1
    ).astype(np.float32)


_NC, _NS, _L = 2, 16, 16
_NW = _NC * _NS
_W = 128


def kernel(x, table):
    B, S = x.shape
    V, D = table.shape
    scale = float(np.sqrt(float(D)))
    pe = jnp.asarray(_positional_encoding(S, D))

    assert B % _W == 0 and D % _L == 0
    n_chunks = (B // _W) * S
    assert n_chunks % _NW == 0
    per_w = n_chunks // _NW
    bblk_per_s = B // _W

    xt = x.T.reshape(-1).astype(jnp.int32)

    mesh = plsc.VectorSubcoreMesh(core_axis_name="c", subcore_axis_name="s")

    @functools.partial(
        pl.kernel,
        mesh=mesh,
        out_type=jax.ShapeDtypeStruct((B, S * D), jnp.float32),
        scratch_types=[
            pltpu.VMEM((S, D), jnp.float32),
            pltpu.VMEM((_W,), jnp.int32),
            pltpu.VMEM((_W, D), jnp.float32),
        ],
    )
    def k(xt_hbm, table_hbm, pe_hbm, out_hbm, pe_v, idx_v, rows_v):
        wid = lax.axis_index("s") * _NC + lax.axis_index("c")
        pltpu.sync_copy(pe_hbm, pe_v)
        base = wid * per_w

        @pl.loop(0, per_w)
        def _(t):
            c = base + t
            s_idx = c // bblk_per_s
            bblk = c % bblk_per_s
            pltpu.sync_copy(xt_hbm.at[pl.ds(c * _W, _W)], idx_v)
            pltpu.sync_copy(table_hbm.at[idx_v], rows_v)
            pe_regs = [pe_v[s_idx, pl.ds(cc * _L, _L)] for cc in range(D // _L)]

            @pl.loop(0, _W)
            def _(i):
                for cc in range(D // _L):
                    sl = pl.ds(cc * _L, _L)
                    rows_v[i, sl] = rows_v[i, sl] * scale + pe_regs[cc]

            pltpu.sync_copy(
                rows_v,
                out_hbm.at[pl.ds(bblk * _W, _W), pl.ds(s_idx * D, D)],
            )

    out = k(xt, table, pe)
    return out.reshape(B, S, D)


# speedup vs baseline: 4.2390x; 1.4964x over previous
import functools

import jax
import jax.numpy as jnp
import numpy as np
from jax import lax
from jax.experimental import pallas as pl
from jax.experimental.pallas import tpu as pltpu
from jax.experimental.pallas import tpu_sc as plsc


def _positional_encoding(length: int, depth: int) -> np.ndarray:
    half = depth // 2
    positions = np.arange(length)[:, np.newaxis]
    depths = np.arange(half)[np.newaxis, :] / half
    angle_rates = 1.0 / (10000.0 ** depths)
    angle_rads = positions * angle_rates
    return np.concatenate(
        [np.sin(angle_rads), np.cos(angle_rads)], axis=-1
    ).astype(np.float32)


_NC, _NS, _L = 2, 16, 16
_NW = _NC * _NS
_W = 128


def kernel(x, table):
    B, S = x.shape
    V, D = table.shape
    scale = float(np.sqrt(float(D)))
    pe = jnp.asarray(_positional_encoding(S, D))

    assert B % _W == 0 and D % _L == 0
    n_chunks = (B // _W) * S
    assert n_chunks % _NW == 0
    per_w = n_chunks // _NW
    assert per_w >= 4 and per_w % 3 == 2
    bblk_per_s = B // _W

    xt = x.T.reshape(_NW, per_w, _W).astype(jnp.int32)

    mesh = plsc.VectorSubcoreMesh(core_axis_name="c", subcore_axis_name="s")

    @functools.partial(
        pl.kernel,
        mesh=mesh,
        out_type=jax.ShapeDtypeStruct((B, S * D), jnp.float32),
        scratch_types=[
            pltpu.VMEM((S, D), jnp.float32),
            pltpu.VMEM((per_w, _W), jnp.int32),
            pltpu.VMEM((_W, D), jnp.float32),
            pltpu.VMEM((_W, D), jnp.float32),
            pltpu.VMEM((_W, D), jnp.float32),
            pltpu.SemaphoreType.DMA,
            pltpu.SemaphoreType.DMA,
            pltpu.SemaphoreType.DMA,
            pltpu.SemaphoreType.DMA,
            pltpu.SemaphoreType.DMA,
            pltpu.SemaphoreType.DMA,
        ],
    )
    def k(xt_hbm, table_hbm, pe_hbm, out_hbm,
          pe_v, idx_v, r0, r1, r2, g0, g1, g2, o0, o1, o2):
        rows = (r0, r1, r2)
        gsem = (g0, g1, g2)
        osem = (o0, o1, o2)
        wid = lax.axis_index("s") * _NC + lax.axis_index("c")
        base = wid * per_w
        pltpu.sync_copy(pe_hbm, pe_v)
        pltpu.sync_copy(xt_hbm.at[wid], idx_v)

        def gather(t, b):
            return pltpu.make_async_copy(
                table_hbm.at[idx_v.at[t]], rows[b], gsem[b])

        def out_slot(c):
            s_idx = c // bblk_per_s
            bblk = c % bblk_per_s
            return out_hbm.at[
                pl.ds(pl.multiple_of(bblk * _W, _W), _W),
                pl.ds(pl.multiple_of(s_idx * D, D), D),
            ]

        def writeback(c, b):
            return pltpu.make_async_copy(rows[b], out_slot(c), osem[b])

        def compute(c, b):
            s_idx = c // bblk_per_s
            r = rows[b]
            pe_regs = [pe_v[s_idx, pl.ds(cc * _L, _L)] for cc in range(D // _L)]

            @pl.loop(0, _W)
            def _(i):
                for cc in range(D // _L):
                    sl = pl.ds(cc * _L, _L)
                    r[i, sl] = r[i, sl] * scale + pe_regs[cc]

        def body(t, b, issue_next, first=False):
            gather(t, b).wait()
            compute(base + t, b)
            pb = (b + 2) % 3

            def _wait_prev():
                writeback(base + t - 1, pb).wait()

            if first:
                pl.when(t >= 1)(_wait_prev)
            else:
                _wait_prev()

            if issue_next:
                gather(t + 2, pb).start()
            writeback(base + t, b).start()

        gather(0, 0).start()
        gather(1, 1).start()

        @pl.loop(0, per_w - 2, step=3)
        def _(t):
            body(t, 0, True, first=True)
            body(t + 1, 1, True)
            body(t + 2, 2, True)

        body(per_w - 2, (per_w - 2) % 3, False)
        body(per_w - 1, (per_w - 1) % 3, False)
        writeback(base + per_w - 1, (per_w - 1) % 3).wait()

    out = k(xt, table, pe)
    return out.reshape(B, S, D)
